# baseline (speedup 1.0000x reference)
import functools

import jax
import jax.numpy as jnp
from jax import lax
from jax.experimental import pallas as pl
from jax.experimental.pallas import tpu as pltpu
from jax.experimental.pallas import tpu_sc as plsc

_CHUNK = 1024


@functools.lru_cache(maxsize=None)
def _make_gather(N, S, D):
    info = plsc.get_sparse_core_info()
    nc, ns, nl = info.num_cores, info.num_subcores, info.num_lanes
    nw = nc * ns
    chunks_per_s = N // _CHUNK
    n_chunks = S * chunks_per_s
    c_per_w = n_chunks // nw
    groups = _CHUNK // nl
    mesh = plsc.VectorSubcoreMesh(core_axis_name="c", subcore_axis_name="s")

    @functools.partial(
        pl.kernel,
        mesh=mesh,
        out_type=jax.ShapeDtypeStruct((S, D, N), jnp.float32),
        scratch_types=[
            pltpu.VMEM((_CHUNK,), jnp.int32),
            pltpu.VMEM((_CHUNK, D), jnp.float32),
            pltpu.VMEM((D, _CHUNK), jnp.float32),
            pltpu.SemaphoreType.DMA,
            pltpu.SemaphoreType.DMA,
        ],
        compiler_params=pltpu.CompilerParams(
            use_tc_tiling_on_sc=False, needs_layout_passes=False
        ),
    )
    def gather_kernel(idx_hbm, table_hbm, out_hbm, idx_v, rows_v, td_v, gsem, ssem):
        wid = lax.axis_index("s") * nc + lax.axis_index("c")
        c_base = wid * c_per_w

        def body(i, carry):
            c = c_base + i
            s = c // chunks_per_s
            b0 = (c % chunks_per_s) * _CHUNK
            pltpu.sync_copy(idx_hbm.at[pl.ds(c * _CHUNK, _CHUNK)], idx_v)
            pltpu.async_copy(table_hbm.at[idx_v], rows_v, gsem).wait()

            @pl.loop(0, groups)
            def _grp(g):
                rows16 = g * nl + lax.iota(jnp.int32, nl)
                for d in range(D):
                    col16 = jnp.full((nl,), d, jnp.int32)
                    vec = plsc.load_gather(rows_v, [rows16, col16])
                    td_v[d, pl.ds(g * nl, nl)] = vec

            pltpu.async_copy(
                td_v, out_hbm.at[s, :, pl.ds(b0, _CHUNK)], ssem
            ).wait()
            return carry

        lax.fori_loop(0, c_per_w, body, 0)

    return gather_kernel


def kernel(indices, weight):
    n, s = indices.shape
    v, d = weight.shape
    flat_idx = indices.T.reshape(s * n)
    out = _make_gather(n, s, d)(flat_idx, weight)
    return out.transpose(2, 0, 1)


# speedup vs baseline: 1.0516x; 1.0516x over previous
import functools

import jax
import jax.numpy as jnp
from jax import lax
from jax.experimental import pallas as pl
from jax.experimental.pallas import tpu as pltpu
from jax.experimental.pallas import tpu_sc as plsc

_CHUNK = 512
_NBUF = 2


@functools.lru_cache(maxsize=None)
def _make_gather(N, S, D):
    info = plsc.get_sparse_core_info()
    nc, ns, nl = info.num_cores, info.num_subcores, info.num_lanes
    nw = nc * ns
    chunks_per_s = N // _CHUNK
    n_chunks = S * chunks_per_s
    c_per_w = n_chunks // nw
    assert c_per_w % _NBUF == 0
    cg = _CHUNK // nl
    mesh = plsc.VectorSubcoreMesh(core_axis_name="c", subcore_axis_name="s")

    scratch = []
    for _ in range(_NBUF):
        scratch += [
            pltpu.VMEM((_CHUNK,), jnp.int32),
            pltpu.VMEM((_CHUNK, D), jnp.float32),
            pltpu.VMEM((D, _CHUNK), jnp.float32),
            pltpu.SemaphoreType.DMA,
            pltpu.SemaphoreType.DMA,
        ]

    @functools.partial(
        pl.kernel,
        mesh=mesh,
        out_type=jax.ShapeDtypeStruct((S, D, N), jnp.float32),
        scratch_types=scratch,
        compiler_params=pltpu.CompilerParams(
            use_tc_tiling_on_sc=False, needs_layout_passes=False
        ),
    )
    def gather_kernel(idx_hbm, table_hbm, out_hbm, *bufs):
        rings = [tuple(bufs[5 * b : 5 * b + 5]) for b in range(_NBUF)]
        wid = lax.axis_index("s") * nc + lax.axis_index("c")
        c_base = wid * c_per_w
        iota = lax.iota(jnp.int32, nl)

        def stage_in(c, b):
            idx_v, rows_v, _, gsem, _ = rings[b]
            pltpu.sync_copy(idx_hbm.at[pl.ds(c * _CHUNK, _CHUNK)], idx_v)
            pltpu.make_async_copy(table_hbm.at[idx_v], rows_v, gsem).start()

        def out_view(c):
            s = c // chunks_per_s
            b0 = (c % chunks_per_s) * _CHUNK
            return out_hbm.at[s, :, pl.ds(b0, _CHUNK)]

        for b in range(_NBUF):
            stage_in(c_base + b, b)

        def body(ii, carry):
            for bb in range(_NBUF):
                i = ii * _NBUF + bb
                c = c_base + i
                idx_v, rows_v, td_v, gsem, ssem = rings[bb]
                pltpu.make_async_copy(table_hbm.at[idx_v], rows_v, gsem).wait()

                @pl.when(i >= _NBUF)
                def _():
                    pltpu.make_async_copy(td_v, out_view(c - _NBUF), ssem).wait()

                @pl.loop(0, cg, unroll=4)
                def _grp(g):
                    r16 = g * nl + iota
                    for d in range(D):
                        vec = plsc.load_gather(
                            rows_v, [r16, jnp.full((nl,), d, jnp.int32)]
                        )
                        td_v[d, pl.ds(g * nl, nl)] = vec

                pltpu.make_async_copy(td_v, out_view(c), ssem).start()

                @pl.when(i + _NBUF < c_per_w)
                def _():
                    stage_in(c + _NBUF, bb)

            return carry

        lax.fori_loop(0, c_per_w // _NBUF, body, 0)

        for b in range(_NBUF):
            _, _, td_v, _, ssem = rings[b]
            pltpu.make_async_copy(
                td_v, out_hbm.at[0, :, pl.ds(0, _CHUNK)], ssem
            ).wait()

    return gather_kernel


def kernel(indices, weight):
    n, s = indices.shape
    v, d = weight.shape
    flat_idx = indices.T.reshape(s * n)
    out = _make_gather(n, s, d)(flat_idx, weight)
    return out.transpose(2, 0, 1)


# speedup vs baseline: 1.3905x; 1.3222x over previous
import functools

import jax
import jax.numpy as jnp
from jax import lax
from jax.experimental import pallas as pl
from jax.experimental.pallas import tpu as pltpu
from jax.experimental.pallas import tpu_sc as plsc

_CHUNK = 512
_NBUF = 2


@functools.lru_cache(maxsize=None)
def _make_gather(N, S, D):
    info = plsc.get_sparse_core_info()
    nc, ns = info.num_cores, info.num_subcores
    nw = nc * ns
    chunks_per_s = N // _CHUNK
    n_chunks = S * chunks_per_s
    c_per_w = n_chunks // nw
    assert c_per_w % _NBUF == 0
    mesh = plsc.VectorSubcoreMesh(core_axis_name="c", subcore_axis_name="s")

    scratch = []
    for _ in range(_NBUF):
        scratch += [
            pltpu.VMEM((_CHUNK,), jnp.int32),
            pltpu.VMEM((_CHUNK, D), jnp.float32),
            pltpu.SemaphoreType.DMA,
            pltpu.SemaphoreType.DMA,
        ]

    @functools.partial(
        pl.kernel,
        mesh=mesh,
        out_type=jax.ShapeDtypeStruct((S, N, D), jnp.float32),
        scratch_types=scratch,
        compiler_params=pltpu.CompilerParams(use_tc_tiling_on_sc=False),
    )
    def gather_kernel(idx_hbm, table_hbm, out_hbm, *bufs):
        rings = [tuple(bufs[4 * b : 4 * b + 4]) for b in range(_NBUF)]
        wid = lax.axis_index("s") * nc + lax.axis_index("c")
        c_base = wid * c_per_w

        def stage_in(c, b):
            idx_v, rows_v, gsem, _ = rings[b]
            pltpu.sync_copy(idx_hbm.at[pl.ds(c * _CHUNK, _CHUNK)], idx_v)
            pltpu.make_async_copy(table_hbm.at[idx_v], rows_v, gsem).start()

        def out_view(c):
            s = c // chunks_per_s
            b0 = (c % chunks_per_s) * _CHUNK
            return out_hbm.at[s, pl.ds(b0, _CHUNK), :]

        for b in range(_NBUF):
            stage_in(c_base + b, b)

        def body(ii, carry):
            for bb in range(_NBUF):
                i = ii * _NBUF + bb
                c = c_base + i
                idx_v, rows_v, gsem, ssem = rings[bb]
                pltpu.make_async_copy(table_hbm.at[idx_v], rows_v, gsem).wait()
                pltpu.make_async_copy(rows_v, out_view(c), ssem).start()

                @pl.when(i + _NBUF < c_per_w)
                def _():
                    pltpu.sync_copy(
                        idx_hbm.at[pl.ds((c + _NBUF) * _CHUNK, _CHUNK)], idx_v
                    )
                    pltpu.make_async_copy(rows_v, out_view(c), ssem).wait()
                    pltpu.make_async_copy(
                        table_hbm.at[idx_v], rows_v, gsem
                    ).start()

            return carry

        lax.fori_loop(0, c_per_w // _NBUF, body, 0)

        for b in range(_NBUF):
            _, rows_v, _, ssem = rings[b]
            pltpu.make_async_copy(
                rows_v, out_hbm.at[0, pl.ds(0, _CHUNK), :], ssem
            ).wait()

    return gather_kernel


def kernel(indices, weight):
    n, s = indices.shape
    v, d = weight.shape
    flat_idx = indices.T.reshape(s * n)
    out = _make_gather(n, s, d)(flat_idx, weight)
    return out.transpose(1, 0, 2)
